# Initial kernel scaffold; baseline (speedup 1.0000x reference)
import functools

import numpy as np
import jax
import jax.numpy as jnp
from jax import lax
from jax.experimental import pallas as pl
from jax.experimental.pallas import tpu as pltpu
from jax.experimental.pallas import tpu_sc as plsc

ALPHA = 0.1
THETA = 0.5

NTILE = 16
NCORE = 2
CH = 128
FH = 64


def _rsqrt16(x):
    i = plsc.bitcast(x, jnp.int32)
    i = jnp.int32(0x5F3759DF) - lax.shift_right_logical(i, 1)
    y = plsc.bitcast(i, jnp.float32)
    for _ in range(3):
        y = y * (1.5 - 0.5 * x * y * y)
    return y


def _lin0_body(x_ref, w_ref, b_ref, o_ref):
    acc = jnp.dot(x_ref[...], w_ref[...], preferred_element_type=jnp.float32)
    o_ref[...] = jnp.maximum(acc + b_ref[...], 0.0)


def _lin0(x, W0, b0):
    n, in_f = x.shape
    hid = W0.shape[1]
    bn = 2000
    grid = n // bn
    return pl.pallas_call(
        _lin0_body,
        grid=(grid,),
        in_specs=[
            pl.BlockSpec((bn, in_f), lambda i: (i, 0)),
            pl.BlockSpec((in_f, hid), lambda i: (0, 0)),
            pl.BlockSpec((1, hid), lambda i: (0, 0)),
        ],
        out_specs=pl.BlockSpec((bn, hid), lambda i: (i, 0)),
        out_shape=jax.ShapeDtypeStruct((n, hid), jnp.float32),
    )(x, W0, b0.reshape(1, hid))


def _make_mix(beta, last):
    a1 = 1.0 - ALPHA
    a0 = ALPHA
    b1 = 1.0 - beta
    b0c = beta

    if last:
        def body(p_ref, x0_ref, w_ref, wo_ref, bo_ref, o_ref):
            t = a1 * p_ref[...] + a0 * x0_ref[...]
            y = b1 * t + b0c * jnp.dot(t, w_ref[...],
                                       preferred_element_type=jnp.float32)
            r = jnp.maximum(y, 0.0)
            o_ref[...] = jnp.dot(r, wo_ref[...],
                                 preferred_element_type=jnp.float32) + bo_ref[...]
    else:
        def body(p_ref, x0_ref, w_ref, o_ref):
            t = a1 * p_ref[...] + a0 * x0_ref[...]
            y = b1 * t + b0c * jnp.dot(t, w_ref[...],
                                       preferred_element_type=jnp.float32)
            o_ref[...] = jnp.maximum(y, 0.0)
    return body


def _tc_layer(p, x0, Wl, beta, last, W_out=None, b_out=None):
    e, h2 = p.shape
    bn = 1280
    grid = e // bn
    out_f = W_out.shape[1] if last else h2
    blk = lambda i: (i, 0)
    zero = lambda i: (0, 0)
    in_specs = [
        pl.BlockSpec((bn, h2), blk),
        pl.BlockSpec((bn, h2), blk),
        pl.BlockSpec((h2, h2), zero),
    ]
    args = [p, x0, Wl]
    if last:
        in_specs += [pl.BlockSpec((h2, out_f), zero),
                     pl.BlockSpec((1, out_f), zero)]
        args += [W_out, b_out.reshape(1, out_f)]
    return pl.pallas_call(
        _make_mix(beta, last),
        grid=(grid,),
        in_specs=in_specs,
        out_specs=pl.BlockSpec((bn, out_f), blk),
        out_shape=jax.ShapeDtypeStruct((e, out_f), jnp.float32),
    )(*args)


def _sc_setup(h, ef, n, e):
    ept = e // NTILE
    nfull = ept // CH
    rem = ept - nfull * CH
    npt = n // NTILE
    hid = h.shape[1]

    mesh = plsc.VectorSubcoreMesh(core_axis_name="c", subcore_axis_name="s")

    @functools.partial(
        pl.kernel,
        mesh=mesh,
        out_type=[
            jax.ShapeDtypeStruct((e, 2 * hid), jnp.float32),
            jax.ShapeDtypeStruct((e,), jnp.float32),
        ],
        scratch_types=[
            pltpu.VMEM_SHARED((n, hid), jnp.float32),
            pltpu.VMEM_SHARED((n, 16), jnp.float32),
            pltpu.VMEM((npt, hid), jnp.float32),
            pltpu.VMEM((CH, 16), jnp.float32),
            pltpu.VMEM((CH,), jnp.int32),
            pltpu.VMEM((16,), jnp.int32),
            pltpu.VMEM((CH, 16), jnp.float32),
            pltpu.VMEM((ept,), jnp.float32),
            pltpu.VMEM((CH, hid), jnp.float32),
        ],
    )
    def setup(h_hbm, ef_hbm, lx_hbm, dis_hbm, h_sp, deg_sp, stage, ones_v,
              idx128, idxr, dtmp, dis_all, grow):
        core = lax.axis_index("c")
        tid = lax.axis_index("s")
        ebase = tid * ept

        @pl.loop(0, npt)
        def _z(r):
            for f in range(hid // 16):
                stage[r, pl.ds(f * 16, 16)] = jnp.zeros((16,), jnp.float32)

        pltpu.sync_copy(stage.at[:, pl.ds(0, 16)],
                        deg_sp.at[pl.ds(tid * npt, npt)])
        pltpu.sync_copy(h_hbm.at[pl.ds(tid * npt, npt)], stage)
        pltpu.sync_copy(stage, h_sp.at[pl.ds(tid * npt, npt)])
        plsc.subcore_barrier()

        @pl.loop(0, CH)
        def _o(r):
            ones_v[r] = jnp.ones((16,), jnp.float32)

        @pl.loop(0, nfull)
        def _deg(k):
            pltpu.sync_copy(ef_hbm.at[pl.ds(e + ebase + k * CH, CH)], idx128)
            pltpu.sync_copy(ones_v, deg_sp.at[idx128], add=True)

        if rem:
            pltpu.sync_copy(ef_hbm.at[pl.ds(e + ebase + nfull * CH, rem)], idxr)
            pltpu.sync_copy(ones_v.at[pl.ds(0, rem)], deg_sp.at[idxr], add=True)
        plsc.subcore_barrier()

        lanes = jax.lax.iota(jnp.int32, 16)
        zeros16 = jnp.zeros((16,), jnp.int32)

        @pl.loop(0, nfull)
        def _dis(k):
            pltpu.sync_copy(ef_hbm.at[pl.ds(ebase + k * CH, CH)], idx128)
            pltpu.sync_copy(deg_sp.at[idx128], dtmp)
            for j in range(CH // 16):
                d = plsc.load_gather(dtmp, [j * 16 + lanes, zeros16])
                dis_all[pl.ds(k * CH + j * 16, 16)] = _rsqrt16(d + 1.0)

        if rem:
            pltpu.sync_copy(ef_hbm.at[pl.ds(ebase + nfull * CH, rem)], idxr)
            pltpu.sync_copy(deg_sp.at[idxr], dtmp.at[pl.ds(0, rem)])
            for j in range(rem // 16):
                d = plsc.load_gather(dtmp, [j * 16 + lanes, zeros16])
                dis_all[pl.ds(nfull * CH + j * 16, 16)] = _rsqrt16(d + 1.0)

        @pl.when(core == 0)
        def _():
            pltpu.sync_copy(dis_all, dis_hbm.at[pl.ds(ebase, ept)])

        @pl.loop(0, nfull)
        def _lx(k):
            pltpu.sync_copy(ef_hbm.at[pl.ds(core * e + ebase + k * CH, CH)],
                            idx128)
            pltpu.sync_copy(h_sp.at[idx128], grow)
            pltpu.sync_copy(grow,
                            lx_hbm.at[pl.ds(ebase + k * CH, CH),
                                      pl.ds(core * hid, hid)])

        if rem:
            pltpu.sync_copy(ef_hbm.at[pl.ds(core * e + ebase + nfull * CH,
                                            rem)], idxr)
            pltpu.sync_copy(h_sp.at[idxr], grow.at[pl.ds(0, rem)])
            pltpu.sync_copy(grow.at[pl.ds(0, rem)],
                            lx_hbm.at[pl.ds(ebase + nfull * CH, rem),
                                      pl.ds(core * hid, hid)])

    return setup(h, ef)


def _sc_prop(cur, dis, ef, n, e):
    ept = e // NTILE
    nfull = ept // CH
    rem = ept - nfull * CH
    npt = n // NTILE
    h2 = cur.shape[1]

    mesh = plsc.VectorSubcoreMesh(core_axis_name="c", subcore_axis_name="s")

    @functools.partial(
        pl.kernel,
        mesh=mesh,
        out_type=jax.ShapeDtypeStruct((e, h2), jnp.float32),
        scratch_types=[
            pltpu.VMEM_SHARED((n, FH), jnp.float32),
            pltpu.VMEM((npt, FH), jnp.float32),
            pltpu.VMEM((CH, FH), jnp.float32),
            pltpu.VMEM((CH, FH), jnp.float32),
            pltpu.VMEM((CH, FH), jnp.float32),
            pltpu.VMEM((CH,), jnp.float32),
            pltpu.VMEM((CH,), jnp.int32),
            pltpu.VMEM((16,), jnp.int32),
        ],
    )
    def prop(cur_hbm, dis_hbm, ef_hbm, p_hbm, agg_sp, zstage, z_v, u_v, g_v,
             d_v, idx128, idxr):
        core = lax.axis_index("c")
        tid = lax.axis_index("s")
        ebase = tid * ept
        fbase = core * FH

        @pl.loop(0, npt)
        def _z(r):
            for f in range(FH // 16):
                zstage[r, pl.ds(f * 16, 16)] = jnp.zeros((16,), jnp.float32)

        pltpu.sync_copy(zstage, agg_sp.at[pl.ds(tid * npt, npt)])
        plsc.subcore_barrier()

        def scale_rows(nrows):
            @pl.loop(0, nrows)
            def _s(i):
                s = d_v[i]
                for f in range(FH // 16):
                    u_v[i, pl.ds(f * 16, 16)] = s * z_v[i, pl.ds(f * 16, 16)]

        def combine_rows(nrows):
            @pl.loop(0, nrows)
            def _c(i):
                s = d_v[i]
                for f in range(FH // 16):
                    sl = pl.ds(f * 16, 16)
                    u_v[i, sl] = s * (g_v[i, sl] + s * z_v[i, sl])

        @pl.loop(0, nfull)
        def _scat(k):
            base = ebase + k * CH
            pltpu.sync_copy(ef_hbm.at[pl.ds(e + base, CH)], idx128)
            pltpu.sync_copy(dis_hbm.at[pl.ds(base, CH)], d_v)
            pltpu.sync_copy(cur_hbm.at[pl.ds(base, CH), pl.ds(fbase, FH)], z_v)
            scale_rows(CH)
            pltpu.sync_copy(u_v, agg_sp.at[idx128], add=True)

        if rem:
            base = ebase + nfull * CH
            pltpu.sync_copy(ef_hbm.at[pl.ds(e + base, rem)], idxr)
            pltpu.sync_copy(dis_hbm.at[pl.ds(base, rem)], d_v.at[pl.ds(0, rem)])
            pltpu.sync_copy(cur_hbm.at[pl.ds(base, rem), pl.ds(fbase, FH)],
                            z_v.at[pl.ds(0, rem)])
            scale_rows(rem)
            pltpu.sync_copy(u_v.at[pl.ds(0, rem)], agg_sp.at[idxr], add=True)
        plsc.subcore_barrier()

        @pl.loop(0, nfull)
        def _gath(k):
            base = ebase + k * CH
            pltpu.sync_copy(ef_hbm.at[pl.ds(base, CH)], idx128)
            pltpu.sync_copy(dis_hbm.at[pl.ds(base, CH)], d_v)
            pltpu.sync_copy(cur_hbm.at[pl.ds(base, CH), pl.ds(fbase, FH)], z_v)
            pltpu.sync_copy(agg_sp.at[idx128], g_v)
            combine_rows(CH)
            pltpu.sync_copy(u_v, p_hbm.at[pl.ds(base, CH), pl.ds(fbase, FH)])

        if rem:
            base = ebase + nfull * CH
            pltpu.sync_copy(ef_hbm.at[pl.ds(base, rem)], idxr)
            pltpu.sync_copy(dis_hbm.at[pl.ds(base, rem)], d_v.at[pl.ds(0, rem)])
            pltpu.sync_copy(cur_hbm.at[pl.ds(base, rem), pl.ds(fbase, FH)],
                            z_v.at[pl.ds(0, rem)])
            pltpu.sync_copy(agg_sp.at[idxr], g_v.at[pl.ds(0, rem)])
            combine_rows(rem)
            pltpu.sync_copy(u_v.at[pl.ds(0, rem)],
                            p_hbm.at[pl.ds(base, rem), pl.ds(fbase, FH)])

    return prop(cur, dis, ef)


def kernel(x, edge_index, W0, b0, conv_W, W_out, b_out):
    n = x.shape[0]
    e = edge_index.shape[1]
    num_layers = conv_W.shape[0]

    ef = edge_index.reshape(-1).astype(jnp.int32)

    h = _lin0(x, W0, b0)
    lx, dis = _sc_setup(h, ef, n, e)

    cur = lx
    for l in range(num_layers):
        beta = float(np.log(THETA / (l + 1) + 1.0))
        p = _sc_prop(cur, dis, ef, n, e)
        last = l == num_layers - 1
        if last:
            cur = _tc_layer(p, lx, conv_W[l], beta, True, W_out, b_out)
        else:
            cur = _tc_layer(p, lx, conv_W[l], beta, False)
    return cur


# speedup vs baseline: 2.2320x; 2.2320x over previous
import functools

import numpy as np
import jax
import jax.numpy as jnp
from jax import lax
from jax.experimental import pallas as pl
from jax.experimental.pallas import tpu as pltpu
from jax.experimental.pallas import tpu_sc as plsc

ALPHA = 0.1
THETA = 0.5

NTILE = 16
NCORE = 2
CH = 128
FH = 64


def _rsqrt16(x):
    i = plsc.bitcast(x, jnp.int32)
    i = jnp.int32(0x5F3759DF) - lax.shift_right_logical(i, 1)
    y = plsc.bitcast(i, jnp.float32)
    for _ in range(3):
        y = y * (1.5 - 0.5 * x * y * y)
    return y


def _lin0_body(x_ref, w_ref, b_ref, o_ref):
    acc = jnp.dot(x_ref[...], w_ref[...], preferred_element_type=jnp.float32)
    o_ref[...] = jnp.maximum(acc + b_ref[...], 0.0)


def _lin0(x, W0, b0):
    n, in_f = x.shape
    hid = W0.shape[1]
    bn = 2000
    grid = n // bn
    return pl.pallas_call(
        _lin0_body,
        grid=(grid,),
        in_specs=[
            pl.BlockSpec((bn, in_f), lambda i: (i, 0)),
            pl.BlockSpec((in_f, hid), lambda i: (0, 0)),
            pl.BlockSpec((1, hid), lambda i: (0, 0)),
        ],
        out_specs=pl.BlockSpec((bn, hid), lambda i: (i, 0)),
        out_shape=jax.ShapeDtypeStruct((n, hid), jnp.float32),
    )(x, W0, b0.reshape(1, hid))


def _make_mix(beta, last):
    a1 = 1.0 - ALPHA
    a0 = ALPHA
    b1 = 1.0 - beta
    b0c = beta

    if last:
        def body(p_ref, x0_ref, w_ref, wo_ref, bo_ref, o_ref):
            t = a1 * p_ref[...] + a0 * x0_ref[...]
            y = b1 * t + b0c * jnp.dot(t, w_ref[...],
                                       preferred_element_type=jnp.float32)
            r = jnp.maximum(y, 0.0)
            o_ref[...] = jnp.dot(r, wo_ref[...],
                                 preferred_element_type=jnp.float32) + bo_ref[...]
    else:
        def body(p_ref, x0_ref, w_ref, o_ref):
            t = a1 * p_ref[...] + a0 * x0_ref[...]
            y = b1 * t + b0c * jnp.dot(t, w_ref[...],
                                       preferred_element_type=jnp.float32)
            o_ref[...] = jnp.maximum(y, 0.0)
    return body


def _tc_layer(p, x0, Wl, beta, last, W_out=None, b_out=None):
    e, h2 = p.shape
    bn = 1280
    grid = e // bn
    out_f = W_out.shape[1] if last else h2
    blk = lambda i: (i, 0)
    zero = lambda i: (0, 0)
    in_specs = [
        pl.BlockSpec((bn, h2), blk),
        pl.BlockSpec((bn, h2), blk),
        pl.BlockSpec((h2, h2), zero),
    ]
    args = [p, x0, Wl]
    if last:
        in_specs += [pl.BlockSpec((h2, out_f), zero),
                     pl.BlockSpec((1, out_f), zero)]
        args += [W_out, b_out.reshape(1, out_f)]
    return pl.pallas_call(
        _make_mix(beta, last),
        grid=(grid,),
        in_specs=in_specs,
        out_specs=pl.BlockSpec((bn, out_f), blk),
        out_shape=jax.ShapeDtypeStruct((e, out_f), jnp.float32),
    )(*args)


def _sc_setup(h, ef, n, e):
    ept = e // NTILE
    nfull = ept // CH
    rem = ept - nfull * CH
    npt = n // NTILE
    hid = h.shape[1]

    mesh = plsc.VectorSubcoreMesh(core_axis_name="c", subcore_axis_name="s")

    @functools.partial(
        pl.kernel,
        mesh=mesh,
        compiler_params=pltpu.CompilerParams(use_tc_tiling_on_sc=False, needs_layout_passes=False),
        out_type=[
            jax.ShapeDtypeStruct((e, 2 * hid), jnp.float32),
            jax.ShapeDtypeStruct((e,), jnp.float32),
        ],
        scratch_types=[
            pltpu.VMEM_SHARED((n, hid), jnp.float32),
            pltpu.VMEM_SHARED((n, 16), jnp.float32),
            pltpu.VMEM((npt, hid), jnp.float32),
            pltpu.VMEM((CH, 16), jnp.float32),
            pltpu.VMEM((CH,), jnp.int32),
            pltpu.VMEM((16,), jnp.int32),
            pltpu.VMEM((CH, 16), jnp.float32),
            pltpu.VMEM((ept,), jnp.float32),
            pltpu.VMEM((CH, hid), jnp.float32),
        ],
    )
    def setup(h_hbm, ef_hbm, lx_hbm, dis_hbm, h_sp, deg_sp, stage, ones_v,
              idx128, idxr, dtmp, dis_all, grow):
        core = lax.axis_index("c")
        tid = lax.axis_index("s")
        ebase = tid * ept

        @pl.loop(0, npt)
        def _z(r):
            for f in range(hid // 16):
                stage[r, pl.ds(f * 16, 16)] = jnp.zeros((16,), jnp.float32)

        pltpu.sync_copy(stage.at[:, pl.ds(0, 16)],
                        deg_sp.at[pl.ds(tid * npt, npt)])
        pltpu.sync_copy(h_hbm.at[pl.ds(tid * npt, npt)], stage)
        pltpu.sync_copy(stage, h_sp.at[pl.ds(tid * npt, npt)])
        plsc.subcore_barrier()

        @pl.loop(0, CH)
        def _o(r):
            ones_v[r] = jnp.ones((16,), jnp.float32)

        @pl.loop(0, nfull)
        def _deg(k):
            pltpu.sync_copy(ef_hbm.at[pl.ds(e + ebase + k * CH, CH)], idx128)
            pltpu.sync_copy(ones_v, deg_sp.at[idx128], add=True)

        if rem:
            pltpu.sync_copy(ef_hbm.at[pl.ds(e + ebase + nfull * CH, rem)], idxr)
            pltpu.sync_copy(ones_v.at[pl.ds(0, rem)], deg_sp.at[idxr], add=True)
        plsc.subcore_barrier()

        lanes = jax.lax.iota(jnp.int32, 16)
        zeros16 = jnp.zeros((16,), jnp.int32)

        @pl.loop(0, nfull)
        def _dis(k):
            pltpu.sync_copy(ef_hbm.at[pl.ds(ebase + k * CH, CH)], idx128)
            pltpu.sync_copy(deg_sp.at[idx128], dtmp)
            for j in range(CH // 16):
                d = plsc.load_gather(dtmp, [j * 16 + lanes, zeros16])
                dis_all[pl.ds(k * CH + j * 16, 16)] = _rsqrt16(d + 1.0)

        if rem:
            pltpu.sync_copy(ef_hbm.at[pl.ds(ebase + nfull * CH, rem)], idxr)
            pltpu.sync_copy(deg_sp.at[idxr], dtmp.at[pl.ds(0, rem)])
            for j in range(rem // 16):
                d = plsc.load_gather(dtmp, [j * 16 + lanes, zeros16])
                dis_all[pl.ds(nfull * CH + j * 16, 16)] = _rsqrt16(d + 1.0)

        @pl.when(core == 0)
        def _():
            pltpu.sync_copy(dis_all, dis_hbm.at[pl.ds(ebase, ept)])

        @pl.loop(0, nfull)
        def _lx(k):
            pltpu.sync_copy(ef_hbm.at[pl.ds(core * e + ebase + k * CH, CH)],
                            idx128)
            pltpu.sync_copy(h_sp.at[idx128], grow)
            pltpu.sync_copy(grow,
                            lx_hbm.at[pl.ds(ebase + k * CH, CH),
                                      pl.ds(core * hid, hid)])

        if rem:
            pltpu.sync_copy(ef_hbm.at[pl.ds(core * e + ebase + nfull * CH,
                                            rem)], idxr)
            pltpu.sync_copy(h_sp.at[idxr], grow.at[pl.ds(0, rem)])
            pltpu.sync_copy(grow.at[pl.ds(0, rem)],
                            lx_hbm.at[pl.ds(ebase + nfull * CH, rem),
                                      pl.ds(core * hid, hid)])

    return setup(h, ef)


def _sc_prop(cur, dis, ef, n, e):
    ept = e // NTILE
    nfull = ept // CH
    rem = ept - nfull * CH
    npt = n // NTILE
    h2 = cur.shape[1]

    mesh = plsc.VectorSubcoreMesh(core_axis_name="c", subcore_axis_name="s")

    @functools.partial(
        pl.kernel,
        mesh=mesh,
        compiler_params=pltpu.CompilerParams(use_tc_tiling_on_sc=False, needs_layout_passes=False),
        out_type=jax.ShapeDtypeStruct((e, h2), jnp.float32),
        scratch_types=[
            pltpu.VMEM_SHARED((n, FH), jnp.float32),
            pltpu.VMEM((npt, FH), jnp.float32),
            pltpu.VMEM((CH, FH), jnp.float32),
            pltpu.VMEM((CH, FH), jnp.float32),
            pltpu.VMEM((CH, FH), jnp.float32),
            pltpu.VMEM((CH,), jnp.float32),
            pltpu.VMEM((CH,), jnp.int32),
            pltpu.VMEM((16,), jnp.int32),
        ],
    )
    def prop(cur_hbm, dis_hbm, ef_hbm, p_hbm, agg_sp, zstage, z_v, u_v, g_v,
             d_v, idx128, idxr):
        core = lax.axis_index("c")
        tid = lax.axis_index("s")
        ebase = tid * ept
        fbase = core * FH

        @pl.loop(0, npt)
        def _z(r):
            for f in range(FH // 16):
                zstage[r, pl.ds(f * 16, 16)] = jnp.zeros((16,), jnp.float32)

        pltpu.sync_copy(zstage, agg_sp.at[pl.ds(tid * npt, npt)])
        plsc.subcore_barrier()

        def scale_rows(nrows):
            @pl.loop(0, nrows // 16)
            def _s(b):
                dvec = d_v[pl.ds(b * 16, 16)]
                for j in range(16):
                    s = dvec[j]
                    i = b * 16 + j
                    for f in range(FH // 16):
                        sl = pl.ds(f * 16, 16)
                        u_v[i, sl] = s * z_v[i, sl]

        def combine_rows(nrows):
            @pl.loop(0, nrows // 16)
            def _c(b):
                dvec = d_v[pl.ds(b * 16, 16)]
                for j in range(16):
                    s = dvec[j]
                    i = b * 16 + j
                    for f in range(FH // 16):
                        sl = pl.ds(f * 16, 16)
                        u_v[i, sl] = s * (g_v[i, sl] + s * z_v[i, sl])

        @pl.loop(0, nfull)
        def _scat(k):
            base = ebase + k * CH
            pltpu.sync_copy(ef_hbm.at[pl.ds(e + base, CH)], idx128)
            pltpu.sync_copy(dis_hbm.at[pl.ds(base, CH)], d_v)
            pltpu.sync_copy(cur_hbm.at[pl.ds(base, CH), pl.ds(fbase, FH)], z_v)
            scale_rows(CH)
            pltpu.sync_copy(u_v, agg_sp.at[idx128], add=True)

        if rem:
            base = ebase + nfull * CH
            pltpu.sync_copy(ef_hbm.at[pl.ds(e + base, rem)], idxr)
            pltpu.sync_copy(dis_hbm.at[pl.ds(base, rem)], d_v.at[pl.ds(0, rem)])
            pltpu.sync_copy(cur_hbm.at[pl.ds(base, rem), pl.ds(fbase, FH)],
                            z_v.at[pl.ds(0, rem)])
            scale_rows(rem)
            pltpu.sync_copy(u_v.at[pl.ds(0, rem)], agg_sp.at[idxr], add=True)
        plsc.subcore_barrier()

        @pl.loop(0, nfull)
        def _gath(k):
            base = ebase + k * CH
            pltpu.sync_copy(ef_hbm.at[pl.ds(base, CH)], idx128)
            pltpu.sync_copy(dis_hbm.at[pl.ds(base, CH)], d_v)
            pltpu.sync_copy(cur_hbm.at[pl.ds(base, CH), pl.ds(fbase, FH)], z_v)
            pltpu.sync_copy(agg_sp.at[idx128], g_v)
            combine_rows(CH)
            pltpu.sync_copy(u_v, p_hbm.at[pl.ds(base, CH), pl.ds(fbase, FH)])

        if rem:
            base = ebase + nfull * CH
            pltpu.sync_copy(ef_hbm.at[pl.ds(base, rem)], idxr)
            pltpu.sync_copy(dis_hbm.at[pl.ds(base, rem)], d_v.at[pl.ds(0, rem)])
            pltpu.sync_copy(cur_hbm.at[pl.ds(base, rem), pl.ds(fbase, FH)],
                            z_v.at[pl.ds(0, rem)])
            pltpu.sync_copy(agg_sp.at[idxr], g_v.at[pl.ds(0, rem)])
            combine_rows(rem)
            pltpu.sync_copy(u_v.at[pl.ds(0, rem)],
                            p_hbm.at[pl.ds(base, rem), pl.ds(fbase, FH)])

    return prop(cur, dis, ef)


def kernel(x, edge_index, W0, b0, conv_W, W_out, b_out):
    n = x.shape[0]
    e = edge_index.shape[1]
    num_layers = conv_W.shape[0]

    ef = edge_index.reshape(-1).astype(jnp.int32)

    h = _lin0(x, W0, b0)
    lx, dis = _sc_setup(h, ef, n, e)

    cur = lx
    for l in range(num_layers):
        beta = float(np.log(THETA / (l + 1) + 1.0))
        p = _sc_prop(cur, dis, ef, n, e)
        last = l == num_layers - 1
        if last:
            cur = _tc_layer(p, lx, conv_W[l], beta, True, W_out, b_out)
        else:
            cur = _tc_layer(p, lx, conv_W[l], beta, False)
    return cur


# speedup vs baseline: 4.0534x; 1.8160x over previous
import functools

import numpy as np
import jax
import jax.numpy as jnp
from jax import lax
from jax.experimental import pallas as pl
from jax.experimental.pallas import tpu as pltpu
from jax.experimental.pallas import tpu_sc as plsc

ALPHA = 0.1
THETA = 0.5

NTILE = 16
NCORE = 2
CH = 128
FH = 64


def _rsqrt16(x):
    i = plsc.bitcast(x, jnp.int32)
    i = jnp.int32(0x5F3759DF) - lax.shift_right_logical(i, 1)
    y = plsc.bitcast(i, jnp.float32)
    for _ in range(3):
        y = y * (1.5 - 0.5 * x * y * y)
    return y


def _lin0_body(x_ref, w_ref, b_ref, o_ref):
    acc = jnp.dot(x_ref[...], w_ref[...], preferred_element_type=jnp.float32)
    o_ref[...] = jnp.maximum(acc + b_ref[...], 0.0)


def _lin0(x, W0, b0):
    n, in_f = x.shape
    hid = W0.shape[1]
    bn = 2000
    grid = n // bn
    return pl.pallas_call(
        _lin0_body,
        grid=(grid,),
        in_specs=[
            pl.BlockSpec((bn, in_f), lambda i: (i, 0)),
            pl.BlockSpec((in_f, hid), lambda i: (0, 0)),
            pl.BlockSpec((1, hid), lambda i: (0, 0)),
        ],
        out_specs=pl.BlockSpec((bn, hid), lambda i: (i, 0)),
        out_shape=jax.ShapeDtypeStruct((n, hid), jnp.float32),
    )(x, W0, b0.reshape(1, hid))


def _make_mix(beta, last):
    a1 = 1.0 - ALPHA
    a0 = ALPHA
    b1 = 1.0 - beta
    b0c = beta

    if last:
        def body(p_ref, x0_ref, w_ref, wo_ref, bo_ref, o_ref):
            t = a1 * p_ref[...] + a0 * x0_ref[...]
            y = b1 * t + b0c * jnp.dot(t, w_ref[...],
                                       preferred_element_type=jnp.float32)
            r = jnp.maximum(y, 0.0)
            o_ref[...] = jnp.dot(r, wo_ref[...],
                                 preferred_element_type=jnp.float32) + bo_ref[...]
    else:
        def body(p_ref, x0_ref, w_ref, o_ref):
            t = a1 * p_ref[...] + a0 * x0_ref[...]
            y = b1 * t + b0c * jnp.dot(t, w_ref[...],
                                       preferred_element_type=jnp.float32)
            o_ref[...] = jnp.maximum(y, 0.0)
    return body


def _tc_layer(p, x0, Wl, beta, last, W_out=None, b_out=None):
    e, h2 = p.shape
    bn = 1280
    grid = e // bn
    out_f = W_out.shape[1] if last else h2
    blk = lambda i: (i, 0)
    zero = lambda i: (0, 0)
    in_specs = [
        pl.BlockSpec((bn, h2), blk),
        pl.BlockSpec((bn, h2), blk),
        pl.BlockSpec((h2, h2), zero),
    ]
    args = [p, x0, Wl]
    if last:
        in_specs += [pl.BlockSpec((h2, out_f), zero),
                     pl.BlockSpec((1, out_f), zero)]
        args += [W_out, b_out.reshape(1, out_f)]
    return pl.pallas_call(
        _make_mix(beta, last),
        grid=(grid,),
        in_specs=in_specs,
        out_specs=pl.BlockSpec((bn, out_f), blk),
        out_shape=jax.ShapeDtypeStruct((e, out_f), jnp.float32),
    )(*args)


def _sc_setup(h, ef, n, e):
    ept = e // NTILE
    nfull = ept // CH
    rem = ept - nfull * CH
    npt = n // NTILE
    hid = h.shape[1]

    mesh = plsc.VectorSubcoreMesh(core_axis_name="c", subcore_axis_name="s")

    @functools.partial(
        pl.kernel,
        mesh=mesh,
        compiler_params=pltpu.CompilerParams(use_tc_tiling_on_sc=False, needs_layout_passes=False),
        out_type=[
            jax.ShapeDtypeStruct((e, 2 * hid), jnp.float32),
            jax.ShapeDtypeStruct((e,), jnp.float32),
        ],
        scratch_types=[
            pltpu.VMEM_SHARED((n, hid), jnp.float32),
            pltpu.VMEM_SHARED((n, 16), jnp.float32),
            pltpu.VMEM((npt, hid), jnp.float32),
            pltpu.VMEM((CH, 16), jnp.float32),
            pltpu.VMEM((CH,), jnp.int32),
            pltpu.VMEM((16,), jnp.int32),
            pltpu.VMEM((CH, 16), jnp.float32),
            pltpu.VMEM((ept,), jnp.float32),
            pltpu.VMEM((CH, hid), jnp.float32),
        ],
    )
    def setup(h_hbm, ef_hbm, lx_hbm, dis_hbm, h_sp, deg_sp, stage, ones_v,
              idx128, idxr, dtmp, dis_all, grow):
        core = lax.axis_index("c")
        tid = lax.axis_index("s")
        ebase = tid * ept

        @pl.loop(0, npt)
        def _z(r):
            for f in range(hid // 16):
                stage[r, pl.ds(f * 16, 16)] = jnp.zeros((16,), jnp.float32)

        pltpu.sync_copy(stage.at[:, pl.ds(0, 16)],
                        deg_sp.at[pl.ds(tid * npt, npt)])
        pltpu.sync_copy(h_hbm.at[pl.ds(tid * npt, npt)], stage)
        pltpu.sync_copy(stage, h_sp.at[pl.ds(tid * npt, npt)])
        plsc.subcore_barrier()

        @pl.loop(0, CH)
        def _o(r):
            ones_v[r] = jnp.ones((16,), jnp.float32)

        @pl.loop(0, nfull)
        def _deg(k):
            pltpu.sync_copy(ef_hbm.at[pl.ds(e + ebase + k * CH, CH)], idx128)
            pltpu.sync_copy(ones_v, deg_sp.at[idx128], add=True)

        if rem:
            pltpu.sync_copy(ef_hbm.at[pl.ds(e + ebase + nfull * CH, rem)], idxr)
            pltpu.sync_copy(ones_v.at[pl.ds(0, rem)], deg_sp.at[idxr], add=True)
        plsc.subcore_barrier()

        lanes = jax.lax.iota(jnp.int32, 16)
        zeros16 = jnp.zeros((16,), jnp.int32)

        @pl.loop(0, nfull)
        def _dis(k):
            pltpu.sync_copy(ef_hbm.at[pl.ds(ebase + k * CH, CH)], idx128)
            pltpu.sync_copy(deg_sp.at[idx128], dtmp)
            for j in range(CH // 16):
                d = plsc.load_gather(dtmp, [j * 16 + lanes, zeros16])
                dis_all[pl.ds(k * CH + j * 16, 16)] = _rsqrt16(d + 1.0)

        if rem:
            pltpu.sync_copy(ef_hbm.at[pl.ds(ebase + nfull * CH, rem)], idxr)
            pltpu.sync_copy(deg_sp.at[idxr], dtmp.at[pl.ds(0, rem)])
            for j in range(rem // 16):
                d = plsc.load_gather(dtmp, [j * 16 + lanes, zeros16])
                dis_all[pl.ds(nfull * CH + j * 16, 16)] = _rsqrt16(d + 1.0)

        @pl.when(core == 0)
        def _():
            pltpu.sync_copy(dis_all, dis_hbm.at[pl.ds(ebase, ept)])

        @pl.loop(0, nfull)
        def _lx(k):
            pltpu.sync_copy(ef_hbm.at[pl.ds(core * e + ebase + k * CH, CH)],
                            idx128)
            pltpu.sync_copy(h_sp.at[idx128], grow)
            pltpu.sync_copy(grow,
                            lx_hbm.at[pl.ds(ebase + k * CH, CH),
                                      pl.ds(core * hid, hid)])

        if rem:
            pltpu.sync_copy(ef_hbm.at[pl.ds(core * e + ebase + nfull * CH,
                                            rem)], idxr)
            pltpu.sync_copy(h_sp.at[idxr], grow.at[pl.ds(0, rem)])
            pltpu.sync_copy(grow.at[pl.ds(0, rem)],
                            lx_hbm.at[pl.ds(ebase + nfull * CH, rem),
                                      pl.ds(core * hid, hid)])

    return setup(h, ef)


def _sc_prop(cur, dis, ef, n, e):
    ept = e // NTILE
    nfull = ept // CH
    rem = ept - nfull * CH
    npt = n // NTILE
    h2 = cur.shape[1]
    assert nfull % 2 == 0 and nfull >= 4

    mesh = plsc.VectorSubcoreMesh(core_axis_name="c", subcore_axis_name="s")

    @functools.partial(
        pl.kernel,
        mesh=mesh,
        compiler_params=pltpu.CompilerParams(use_tc_tiling_on_sc=False,
                                             needs_layout_passes=False),
        out_type=jax.ShapeDtypeStruct((e, h2), jnp.float32),
        scratch_types=[
            pltpu.VMEM_SHARED((n, FH), jnp.float32),
            pltpu.VMEM((npt, FH), jnp.float32),
            pltpu.VMEM((CH, FH), jnp.float32),
            pltpu.VMEM((CH, FH), jnp.float32),
            pltpu.VMEM((CH, FH), jnp.float32),
            pltpu.VMEM((CH, FH), jnp.float32),
            pltpu.VMEM((CH, FH), jnp.float32),
            pltpu.VMEM((CH,), jnp.float32),
            pltpu.VMEM((CH,), jnp.float32),
            pltpu.VMEM((CH,), jnp.int32),
            pltpu.VMEM((CH,), jnp.int32),
            pltpu.VMEM((16,), jnp.int32),
            pltpu.SemaphoreType.DMA,
            pltpu.SemaphoreType.DMA,
            pltpu.SemaphoreType.DMA,
            pltpu.SemaphoreType.DMA,
        ],
    )
    def prop(cur_hbm, dis_hbm, ef_hbm, p_hbm, agg_sp, zstage, z0, z1, u0, u1,
             g_v, d0, d1, i0, i1, idxr, lsem0, lsem1, osem0, osem1):
        core = lax.axis_index("c")
        tid = lax.axis_index("s")
        ebase = tid * ept
        fbase = core * FH
        zvs, uvs, dvs, ivs = [z0, z1], [u0, u1], [d0, d1], [i0, i1]
        lsems, osems = [lsem0, lsem1], [osem0, osem1]

        @pl.loop(0, npt)
        def _z(r):
            for f in range(FH // 16):
                zstage[r, pl.ds(f * 16, 16)] = jnp.zeros((16,), jnp.float32)

        pltpu.sync_copy(zstage, agg_sp.at[pl.ds(tid * npt, npt)])
        plsc.subcore_barrier()

        def load_descs(k, b):
            base = ebase + k * CH
            return (
                pltpu.make_async_copy(ef_hbm.at[pl.ds(e + base, CH)], ivs[b],
                                      lsems[b]),
                pltpu.make_async_copy(dis_hbm.at[pl.ds(base, CH)], dvs[b],
                                      lsems[b]),
                pltpu.make_async_copy(
                    cur_hbm.at[pl.ds(base, CH), pl.ds(fbase, FH)], zvs[b],
                    lsems[b]),
            )

        def load_descs2(k, b):
            base = ebase + k * CH
            return (
                pltpu.make_async_copy(ef_hbm.at[pl.ds(base, CH)], ivs[b],
                                      lsems[b]),
                pltpu.make_async_copy(dis_hbm.at[pl.ds(base, CH)], dvs[b],
                                      lsems[b]),
                pltpu.make_async_copy(
                    cur_hbm.at[pl.ds(base, CH), pl.ds(fbase, FH)], zvs[b],
                    lsems[b]),
            )

        def scale_rows(b, dst):
            z_v, d_v = zvs[b], dvs[b]

            @pl.loop(0, CH // 16)
            def _s(g):
                dvec = d_v[pl.ds(g * 16, 16)]
                for j in range(16):
                    s = dvec[j]
                    i = g * 16 + j
                    for f in range(FH // 16):
                        sl = pl.ds(f * 16, 16)
                        dst[i, sl] = s * z_v[i, sl]

        def combine_rows(b):
            z_v, d_v, u_v = zvs[b], dvs[b], uvs[b]

            @pl.loop(0, CH // 16)
            def _c(g):
                dvec = d_v[pl.ds(g * 16, 16)]
                for j in range(16):
                    s = dvec[j]
                    i = g * 16 + j
                    for f in range(FH // 16):
                        sl = pl.ds(f * 16, 16)
                        u_v[i, sl] = s * (g_v[i, sl] + s * z_v[i, sl])

        for b in range(2):
            for dsc in load_descs(b, b):
                dsc.start()

        @pl.loop(0, nfull // 2)
        def _scat(gi):
            for b in range(2):
                k = gi * 2 + b
                for dsc in load_descs(k, b):
                    dsc.wait()
                scale_rows(b, u0)
                pltpu.sync_copy(u0, agg_sp.at[ivs[b]], add=True)

                @pl.when(k + 2 < nfull)
                def _():
                    for dsc in load_descs(k + 2, b):
                        dsc.start()

        if rem:
            base = ebase + nfull * CH
            pltpu.sync_copy(ef_hbm.at[pl.ds(e + base, rem)], idxr)
            pltpu.sync_copy(dis_hbm.at[pl.ds(base, rem)], d0.at[pl.ds(0, rem)])
            pltpu.sync_copy(cur_hbm.at[pl.ds(base, rem), pl.ds(fbase, FH)],
                            z0.at[pl.ds(0, rem)])

            @pl.loop(0, rem // 16)
            def _sr(g):
                dvec = d0[pl.ds(g * 16, 16)]
                for j in range(16):
                    s = dvec[j]
                    i = g * 16 + j
                    for f in range(FH // 16):
                        sl = pl.ds(f * 16, 16)
                        u0[i, sl] = s * z0[i, sl]

            pltpu.sync_copy(u0.at[pl.ds(0, rem)], agg_sp.at[idxr], add=True)
        plsc.subcore_barrier()

        def out_desc(k, b):
            base = ebase + k * CH
            return pltpu.make_async_copy(
                uvs[b], p_hbm.at[pl.ds(base, CH), pl.ds(fbase, FH)], osems[b])

        for b in range(2):
            for dsc in load_descs2(b, b):
                dsc.start()

        @pl.loop(0, nfull // 2)
        def _gath(gi):
            for b in range(2):
                k = gi * 2 + b
                for dsc in load_descs2(k, b):
                    dsc.wait()
                pltpu.sync_copy(agg_sp.at[ivs[b]], g_v)

                @pl.when(gi >= 1)
                def _():
                    out_desc(k, b).wait()

                combine_rows(b)
                out_desc(k, b).start()

                @pl.when(k + 2 < nfull)
                def _():
                    for dsc in load_descs2(k + 2, b):
                        dsc.start()

        for b in range(2):
            out_desc(nfull - 2 + b, b).wait()

        if rem:
            base = ebase + nfull * CH
            pltpu.sync_copy(ef_hbm.at[pl.ds(base, rem)], idxr)
            pltpu.sync_copy(dis_hbm.at[pl.ds(base, rem)], d0.at[pl.ds(0, rem)])
            pltpu.sync_copy(cur_hbm.at[pl.ds(base, rem), pl.ds(fbase, FH)],
                            z0.at[pl.ds(0, rem)])
            pltpu.sync_copy(agg_sp.at[idxr], g_v.at[pl.ds(0, rem)])

            @pl.loop(0, rem // 16)
            def _cr(g):
                dvec = d0[pl.ds(g * 16, 16)]
                for j in range(16):
                    s = dvec[j]
                    i = g * 16 + j
                    for f in range(FH // 16):
                        sl = pl.ds(f * 16, 16)
                        u0[i, sl] = s * (g_v[i, sl] + s * z0[i, sl])

            pltpu.sync_copy(u0.at[pl.ds(0, rem)],
                            p_hbm.at[pl.ds(base, rem), pl.ds(fbase, FH)])

    return prop(cur, dis, ef)


def kernel(x, edge_index, W0, b0, conv_W, W_out, b_out):
    n = x.shape[0]
    e = edge_index.shape[1]
    num_layers = conv_W.shape[0]

    ef = edge_index.reshape(-1).astype(jnp.int32)

    h = _lin0(x, W0, b0)
    lx, dis = _sc_setup(h, ef, n, e)

    cur = lx
    for l in range(num_layers):
        beta = float(np.log(THETA / (l + 1) + 1.0))
        p = _sc_prop(cur, dis, ef, n, e)
        last = l == num_layers - 1
        if last:
            cur = _tc_layer(p, lx, conv_W[l], beta, True, W_out, b_out)
        else:
            cur = _tc_layer(p, lx, conv_W[l], beta, False)
    return cur
